# Initial kernel scaffold; baseline (speedup 1.0000x reference)
import functools

import jax
import jax.numpy as jnp
from jax import lax
from jax.experimental import pallas as pl
from jax.experimental.pallas import tpu as pltpu
from jax.experimental.pallas import tpu_sc as plsc

N_NODES = 10000
F = 128
N_EDGES = 320000

NC = 2
NS = 16
NW = NC * NS
EPT = N_EDGES // NW
WIN = 80
NWIN = EPT // WIN
ZROWS = 125
ROWS_PT = N_NODES // NS
DEG_PAD = 10240
DEG_PT = DEG_PAD // NS

_mesh = plsc.VectorSubcoreMesh(core_axis_name="c", subcore_axis_name="s")


@functools.partial(
    pl.kernel,
    out_type=jax.ShapeDtypeStruct((NC, DEG_PAD), jnp.float32),
    mesh=_mesh,
    scratch_types=[
        pltpu.VMEM((EPT,), jnp.int32),
        pltpu.VMEM((DEG_PAD,), jnp.float32),
        pltpu.VMEM((NS, DEG_PT), jnp.float32),
        pltpu.VMEM_SHARED((NS, DEG_PAD), jnp.float32),
    ],
)
def _deg_kernel(dst_hbm, deg_out, dst_v, hist_v, stripe_v, stage_sh):
    c = lax.axis_index("c")
    s = lax.axis_index("s")
    g = c * NS + s
    pltpu.sync_copy(dst_hbm.at[pl.ds(g * EPT, EPT)], dst_v)

    zero16 = jnp.zeros((16,), jnp.float32)
    ones16 = jnp.ones((16,), jnp.float32)

    @pl.loop(0, DEG_PAD, step=16)
    def _(i):
        hist_v[pl.ds(i, 16)] = zero16

    @pl.loop(0, EPT, step=16)
    def _(i):
        plsc.addupdate_scatter(hist_v, [dst_v[pl.ds(i, 16)]], ones16)

    pltpu.sync_copy(hist_v, stage_sh.at[s])
    plsc.subcore_barrier()
    for r in range(NS):
        pltpu.sync_copy(stage_sh.at[r, pl.ds(s * DEG_PT, DEG_PT)], stripe_v.at[r])

    @pl.loop(0, DEG_PT, step=16)
    def _(i):
        acc = stripe_v[0, pl.ds(i, 16)]
        for r in range(1, NS):
            acc = acc + stripe_v[r, pl.ds(i, 16)]
        stripe_v[0, pl.ds(i, 16)] = acc

    pltpu.sync_copy(stripe_v.at[0], deg_out.at[c, pl.ds(s * DEG_PT, DEG_PT)])


@functools.partial(
    pl.kernel,
    out_type=jax.ShapeDtypeStruct((NC, N_NODES, F), jnp.float32),
    mesh=_mesh,
    scratch_types=[
        pltpu.VMEM((NWIN, WIN), jnp.int32),
        pltpu.VMEM((NWIN, WIN), jnp.int32),
        pltpu.VMEM((WIN, F), jnp.float32),
        pltpu.VMEM((WIN, F), jnp.float32),
        pltpu.VMEM((ZROWS, F), jnp.float32),
        pltpu.VMEM_SHARED((N_NODES, F), jnp.float32),
        pltpu.SemaphoreType.DMA,
        pltpu.SemaphoreType.DMA,
    ],
)
def _agg_kernel(hp_hbm, src_hbm, dst_hbm, out_hbm,
                src_v, dst_v, rows_a, rows_b, zbuf, agg_sh, sem_a, sem_b):
    c = lax.axis_index("c")
    s = lax.axis_index("s")
    g = c * NS + s
    pltpu.sync_copy(src_hbm.at[g], src_v)
    pltpu.sync_copy(dst_hbm.at[g], dst_v)

    zero16 = jnp.zeros((16,), jnp.float32)

    @pl.loop(0, ZROWS)
    def _(i):
        for cc in range(F // 16):
            zbuf[i, pl.ds(cc * 16, 16)] = zero16

    for kk in range(ROWS_PT // ZROWS):
        pltpu.sync_copy(zbuf, agg_sh.at[pl.ds(s * ROWS_PT + kk * ZROWS, ZROWS)])
    plsc.subcore_barrier()

    def start(j, buf, sem):
        pltpu.make_async_copy(hp_hbm.at[src_v.at[j]], buf, sem).start()

    def wait(j, buf, sem):
        pltpu.make_async_copy(hp_hbm.at[src_v.at[j]], buf, sem).wait()

    def scat(j, buf):
        pltpu.sync_copy(buf, agg_sh.at[dst_v.at[j]], add=True)

    start(0, rows_a, sem_a)

    @pl.loop(0, NWIN - 1, step=2)
    def _(w0):
        w1 = w0 + 1
        start(w1, rows_b, sem_b)
        wait(w0, rows_a, sem_a)
        scat(w0, rows_a)
        start(w0 + 2, rows_a, sem_a)
        wait(w1, rows_b, sem_b)
        scat(w1, rows_b)

    wait(NWIN - 1, rows_a, sem_a)
    scat(NWIN - 1, rows_a)
    plsc.subcore_barrier()

    for kk in range(ROWS_PT // ZROWS):
        off = s * ROWS_PT + kk * ZROWS
        pltpu.sync_copy(agg_sh.at[pl.ds(off, ZROWS)],
                        out_hbm.at[c, pl.ds(off, ZROWS)])


_R = 1000


def _mm_scale(x, w_t, b, d0, d1):
    def body(x_ref, w_ref, b_ref, d0_ref, d1_ref, o_ref):
        dinv = lax.rsqrt(d0_ref[...] + d1_ref[...] + 1.0)
        o_ref[...] = dinv * (
            jnp.dot(x_ref[...], w_ref[...], preferred_element_type=jnp.float32)
            + b_ref[...])

    return pl.pallas_call(
        body,
        grid=(N_NODES // _R,),
        in_specs=[
            pl.BlockSpec((_R, F), lambda i: (i, 0)),
            pl.BlockSpec((F, F), lambda i: (0, 0)),
            pl.BlockSpec((1, F), lambda i: (0, 0)),
            pl.BlockSpec((_R, 1), lambda i: (i, 0)),
            pl.BlockSpec((_R, 1), lambda i: (i, 0)),
        ],
        out_specs=pl.BlockSpec((_R, F), lambda i: (i, 0)),
        out_shape=jax.ShapeDtypeStruct((N_NODES, F), jnp.float32),
    )(x, w_t, b, d0, d1)


def _relu_comb_mm(a0, a1, hp, w_t, b, d0, d1):
    def body(a0_ref, a1_ref, hp_ref, w_ref, b_ref, d0_ref, d1_ref, o_ref):
        dinv = lax.rsqrt(d0_ref[...] + d1_ref[...] + 1.0)
        sblk = jnp.maximum(dinv * (a0_ref[...] + a1_ref[...] + hp_ref[...]), 0.0)
        o_ref[...] = dinv * (
            jnp.dot(sblk, w_ref[...], preferred_element_type=jnp.float32)
            + b_ref[...])

    return pl.pallas_call(
        body,
        grid=(N_NODES // _R,),
        in_specs=[
            pl.BlockSpec((_R, F), lambda i: (i, 0)),
            pl.BlockSpec((_R, F), lambda i: (i, 0)),
            pl.BlockSpec((_R, F), lambda i: (i, 0)),
            pl.BlockSpec((F, F), lambda i: (0, 0)),
            pl.BlockSpec((1, F), lambda i: (0, 0)),
            pl.BlockSpec((_R, 1), lambda i: (i, 0)),
            pl.BlockSpec((_R, 1), lambda i: (i, 0)),
        ],
        out_specs=pl.BlockSpec((_R, F), lambda i: (i, 0)),
        out_shape=jax.ShapeDtypeStruct((N_NODES, F), jnp.float32),
    )(a0, a1, hp, w_t, b, d0, d1)


def _final_comb(a0, a1, hp, d0, d1):
    def body(a0_ref, a1_ref, hp_ref, d0_ref, d1_ref, o_ref):
        dinv = lax.rsqrt(d0_ref[...] + d1_ref[...] + 1.0)
        o_ref[...] = dinv * (a0_ref[...] + a1_ref[...] + hp_ref[...])

    return pl.pallas_call(
        body,
        grid=(N_NODES // _R,),
        in_specs=[
            pl.BlockSpec((_R, F), lambda i: (i, 0)),
            pl.BlockSpec((_R, F), lambda i: (i, 0)),
            pl.BlockSpec((_R, F), lambda i: (i, 0)),
            pl.BlockSpec((_R, 1), lambda i: (i, 0)),
            pl.BlockSpec((_R, 1), lambda i: (i, 0)),
        ],
        out_specs=pl.BlockSpec((_R, F), lambda i: (i, 0)),
        out_shape=jax.ShapeDtypeStruct((N_NODES, F), jnp.float32),
    )(a0, a1, hp, d0, d1)


def kernel(x, ei, W1, b1, W2, b2):
    ei = ei.astype(jnp.int32)
    src = ei[0].reshape(NW, NWIN, WIN)
    dst = ei[1].reshape(NW, NWIN, WIN)

    deg = _deg_kernel(ei[1])
    d0 = deg[0, :N_NODES].reshape(N_NODES, 1)
    d1 = deg[1, :N_NODES].reshape(N_NODES, 1)

    hp1 = _mm_scale(x, W1.T, b1.reshape(1, F), d0, d1)
    a1 = _agg_kernel(hp1, src, dst)
    hp2 = _relu_comb_mm(a1[0], a1[1], hp1, W2.T, b2.reshape(1, F), d0, d1)
    a2 = _agg_kernel(hp2, src, dst)
    return _final_comb(a2[0], a2[1], hp2, d0, d1)


# speedup vs baseline: 21.9884x; 21.9884x over previous
import dataclasses
import functools

import jax
import jax.numpy as jnp
from jax import lax
from jax.experimental import pallas as pl
from jax.experimental.pallas import tpu as pltpu
from jax.experimental.pallas import tpu_sc as plsc

N_NODES = 10000
F = 128
FH = F // 2
N_EDGES = 320000

NC = 2
NS = 16
EPT = N_EDGES // NS
WIN = 80
NWIN = EPT // WIN
N_PAD = 10240
ZROWS = 128
ROWS_PT = N_PAD // NS
DEG_PAD = 10240
DEG_PT = DEG_PAD // NS

_mesh = plsc.VectorSubcoreMesh(core_axis_name="c", subcore_axis_name="s")

_sc_params = pltpu.CompilerParams(
    needs_layout_passes=False, use_tc_tiling_on_sc=False)


@functools.partial(
    pl.kernel,
    out_type=jax.ShapeDtypeStruct((NC, DEG_PAD), jnp.float32),
    mesh=_mesh,
    scratch_types=[
        pltpu.VMEM((EPT // 2,), jnp.int32),
        pltpu.VMEM((DEG_PAD,), jnp.float32),
        pltpu.VMEM((NS, DEG_PT), jnp.float32),
        pltpu.VMEM_SHARED((NS, DEG_PAD), jnp.float32),
    ],
    compiler_params=_sc_params,
)
def _deg_kernel(dst_hbm, deg_out, dst_v, hist_v, stripe_v, stage_sh):
    c = lax.axis_index("c")
    s = lax.axis_index("s")
    g = c * NS + s
    half = EPT // 2
    pltpu.sync_copy(dst_hbm.at[pl.ds(g * half, half)], dst_v)

    zero16 = jnp.zeros((16,), jnp.float32)
    ones16 = jnp.ones((16,), jnp.float32)

    @pl.loop(0, DEG_PAD, step=16)
    def _(i):
        hist_v[pl.ds(i, 16)] = zero16

    @pl.loop(0, half, step=16)
    def _(i):
        plsc.addupdate_scatter(hist_v, [dst_v[pl.ds(i, 16)]], ones16)

    pltpu.sync_copy(hist_v, stage_sh.at[s])
    plsc.subcore_barrier()
    for r in range(NS):
        pltpu.sync_copy(stage_sh.at[r, pl.ds(s * DEG_PT, DEG_PT)], stripe_v.at[r])

    @pl.loop(0, DEG_PT, step=16)
    def _(i):
        acc = stripe_v[0, pl.ds(i, 16)]
        for r in range(1, NS):
            acc = acc + stripe_v[r, pl.ds(i, 16)]
        stripe_v[0, pl.ds(i, 16)] = acc

    pltpu.sync_copy(stripe_v.at[0], deg_out.at[c, pl.ds(s * DEG_PT, DEG_PT)])


@functools.partial(
    pl.kernel,
    out_type=jax.ShapeDtypeStruct((NC, N_PAD, FH), jnp.float32),
    mesh=_mesh,
    scratch_types=[
        pltpu.VMEM((NWIN, WIN), jnp.int32),
        pltpu.VMEM((NWIN, WIN), jnp.int32),
        pltpu.VMEM((WIN, FH), jnp.float32),
        pltpu.VMEM((WIN, FH), jnp.float32),
        pltpu.VMEM((ZROWS, FH), jnp.float32),
        pltpu.VMEM_SHARED((N_PAD, FH), jnp.float32),
        pltpu.SemaphoreType.DMA,
        pltpu.SemaphoreType.DMA,
    ],
    compiler_params=_sc_params,
)
def _agg_kernel(hp_hbm, src_hbm, dst_hbm, out_hbm,
                src_v, dst_v, rows_a, rows_b, zbuf, agg_sh, sem_a, sem_b):
    c = lax.axis_index("c")
    s = lax.axis_index("s")
    hp_c = hp_hbm.at[c]
    pltpu.sync_copy(src_hbm.at[s], src_v)
    pltpu.sync_copy(dst_hbm.at[s], dst_v)

    zero16 = jnp.zeros((16,), jnp.float32)

    @pl.loop(0, ZROWS)
    def _(i):
        for cc in range(FH // 16):
            zbuf[i, pl.ds(cc * 16, 16)] = zero16

    for kk in range(ROWS_PT // ZROWS):
        pltpu.sync_copy(zbuf, agg_sh.at[pl.ds(s * ROWS_PT + kk * ZROWS, ZROWS)])
    plsc.subcore_barrier()

    def start(j, buf, sem):
        pltpu.make_async_copy(hp_c.at[src_v.at[j]], buf, sem).start()

    def wait(j, buf, sem):
        pltpu.make_async_copy(hp_c.at[src_v.at[j]], buf, sem).wait()

    def scat(j, buf):
        pltpu.sync_copy(buf, agg_sh.at[dst_v.at[j]], add=True)

    start(0, rows_a, sem_a)

    @pl.loop(0, NWIN - 2, step=2)
    def _(w0):
        w1 = w0 + 1
        start(w1, rows_b, sem_b)
        wait(w0, rows_a, sem_a)
        scat(w0, rows_a)
        start(w0 + 2, rows_a, sem_a)
        wait(w1, rows_b, sem_b)
        scat(w1, rows_b)

    start(NWIN - 1, rows_b, sem_b)
    wait(NWIN - 2, rows_a, sem_a)
    scat(NWIN - 2, rows_a)
    wait(NWIN - 1, rows_b, sem_b)
    scat(NWIN - 1, rows_b)
    plsc.subcore_barrier()

    for kk in range(ROWS_PT // ZROWS):
        off = s * ROWS_PT + kk * ZROWS
        pltpu.sync_copy(agg_sh.at[pl.ds(off, ZROWS)],
                        out_hbm.at[c, pl.ds(off, ZROWS)])


_R = 1000


def _mm_scale(x, w_t, b, d0, d1):
    def body(x_ref, w_ref, b_ref, d0_ref, d1_ref, o_ref):
        dinv = lax.rsqrt(d0_ref[...] + d1_ref[...] + 1.0)
        res = dinv * (
            jnp.dot(x_ref[...], w_ref[...], preferred_element_type=jnp.float32)
            + b_ref[...])
        o_ref[0] = res[:, :FH]
        o_ref[1] = res[:, FH:]

    return pl.pallas_call(
        body,
        grid=(N_NODES // _R,),
        in_specs=[
            pl.BlockSpec((_R, F), lambda i: (i, 0)),
            pl.BlockSpec((F, F), lambda i: (0, 0)),
            pl.BlockSpec((1, F), lambda i: (0, 0)),
            pl.BlockSpec((_R, 1), lambda i: (i, 0)),
            pl.BlockSpec((_R, 1), lambda i: (i, 0)),
        ],
        out_specs=pl.BlockSpec((NC, _R, FH), lambda i: (0, i, 0)),
        out_shape=jax.ShapeDtypeStruct((NC, N_NODES, FH), jnp.float32),
    )(x, w_t, b, d0, d1)


def _relu_comb_mm(a_sp, hp_sp, w_t, b, d0, d1):
    def body(a_ref, hp_ref, w_ref, b_ref, d0_ref, d1_ref, o_ref):
        dinv = lax.rsqrt(d0_ref[...] + d1_ref[...] + 1.0)
        s_lo = jnp.maximum(dinv * (a_ref[0] + hp_ref[0]), 0.0)
        s_hi = jnp.maximum(dinv * (a_ref[1] + hp_ref[1]), 0.0)
        sblk = jnp.concatenate([s_lo, s_hi], axis=1)
        res = dinv * (
            jnp.dot(sblk, w_ref[...], preferred_element_type=jnp.float32)
            + b_ref[...])
        o_ref[0] = res[:, :FH]
        o_ref[1] = res[:, FH:]

    return pl.pallas_call(
        body,
        grid=(N_NODES // _R,),
        in_specs=[
            pl.BlockSpec((NC, _R, FH), lambda i: (0, i, 0)),
            pl.BlockSpec((NC, _R, FH), lambda i: (0, i, 0)),
            pl.BlockSpec((F, F), lambda i: (0, 0)),
            pl.BlockSpec((1, F), lambda i: (0, 0)),
            pl.BlockSpec((_R, 1), lambda i: (i, 0)),
            pl.BlockSpec((_R, 1), lambda i: (i, 0)),
        ],
        out_specs=pl.BlockSpec((NC, _R, FH), lambda i: (0, i, 0)),
        out_shape=jax.ShapeDtypeStruct((NC, N_NODES, FH), jnp.float32),
    )(a_sp, hp_sp, w_t, b, d0, d1)


def _final_comb(a_sp, hp_sp, d0, d1):
    def body(a_ref, hp_ref, d0_ref, d1_ref, o_ref):
        dinv = lax.rsqrt(d0_ref[...] + d1_ref[...] + 1.0)
        o_lo = dinv * (a_ref[0] + hp_ref[0])
        o_hi = dinv * (a_ref[1] + hp_ref[1])
        o_ref[...] = jnp.concatenate([o_lo, o_hi], axis=1)

    return pl.pallas_call(
        body,
        grid=(N_NODES // _R,),
        in_specs=[
            pl.BlockSpec((NC, _R, FH), lambda i: (0, i, 0)),
            pl.BlockSpec((NC, _R, FH), lambda i: (0, i, 0)),
            pl.BlockSpec((_R, 1), lambda i: (i, 0)),
            pl.BlockSpec((_R, 1), lambda i: (i, 0)),
        ],
        out_specs=pl.BlockSpec((_R, F), lambda i: (i, 0)),
        out_shape=jax.ShapeDtypeStruct((N_NODES, F), jnp.float32),
    )(a_sp, hp_sp, d0, d1)


def kernel(x, ei, W1, b1, W2, b2):
    ei = ei.astype(jnp.int32)
    src = ei[0].reshape(NS, NWIN, WIN)
    dst = ei[1].reshape(NS, NWIN, WIN)

    deg = _deg_kernel(ei[1])
    d0 = deg[0, :N_NODES].reshape(N_NODES, 1)
    d1 = deg[1, :N_NODES].reshape(N_NODES, 1)

    hp1 = _mm_scale(x, W1.T, b1.reshape(1, F), d0, d1)
    a1 = _agg_kernel(hp1, src, dst)
    hp2 = _relu_comb_mm(a1, hp1, W2.T, b2.reshape(1, F), d0, d1)
    a2 = _agg_kernel(hp2, src, dst)
    return _final_comb(a2, hp2, d0, d1)
